# baseline (speedup 1.0000x reference)
import functools

import jax
import jax.numpy as jnp
from jax import lax
from jax.experimental import pallas as pl
from jax.experimental.pallas import tpu as pltpu
from jax.experimental.pallas import tpu_sc as plsc

_NUM_CORES = 2
_NUM_SUBCORES = 16
_NUM_WORKERS = _NUM_CORES * _NUM_SUBCORES

_BLK = 4096
_PAD_W = 128


def _tc_project_table(table_t, w_pad, b_pad):
    dim, n = table_t.shape
    grid = (n + _BLK - 1) // _BLK

    def body(x_ref, w_ref, b_ref, o_ref):
        o_ref[...] = lax.dot_general(
            x_ref[...],
            w_ref[...],
            dimension_numbers=(((0,), (1,)), ((), ())),
            preferred_element_type=jnp.float32,
            precision=lax.Precision.DEFAULT,
        ) + b_ref[...]

    return pl.pallas_call(
        body,
        grid=(grid,),
        in_specs=[
            pl.BlockSpec((dim, _BLK), lambda i: (0, i)),
            pl.BlockSpec((_PAD_W, dim), lambda i: (0, 0)),
            pl.BlockSpec((1, _PAD_W), lambda i: (0, 0)),
        ],
        out_specs=pl.BlockSpec((_BLK, _PAD_W), lambda i: (i, 0)),
        out_shape=jax.ShapeDtypeStruct((n, _PAD_W), jnp.float32),
        compiler_params=pltpu.CompilerParams(
            dimension_semantics=("parallel",),
        ),
    )(table_t, w_pad, b_pad)


def _sc_gather(projected, idx):
    batch = idx.shape[0]
    width = projected.shape[1]
    b_per_w = batch // _NUM_WORKERS
    mesh = plsc.VectorSubcoreMesh(core_axis_name="c", subcore_axis_name="s")

    @functools.partial(
        pl.kernel,
        mesh=mesh,
        out_type=jax.ShapeDtypeStruct((batch, width), projected.dtype),
        scratch_types=[
            pltpu.VMEM((b_per_w,), jnp.int32),
            pltpu.VMEM((b_per_w, width), projected.dtype),
            pltpu.SemaphoreType.DMA,
        ],
    )
    def gather_kernel(tbl_hbm, idx_hbm, out_hbm, idx_v, rows_v, sem):
        wid = lax.axis_index("s") * _NUM_CORES + lax.axis_index("c")
        base = wid * b_per_w
        pltpu.sync_copy(idx_hbm.at[pl.ds(base, b_per_w)], idx_v)
        pltpu.async_copy(tbl_hbm.at[idx_v], rows_v, sem).wait()
        pltpu.sync_copy(rows_v, out_hbm.at[pl.ds(base, b_per_w)])

    return gather_kernel(projected, idx)


def _tc_slice(gathered, dim):
    batch = gathered.shape[0]

    def body(x_ref, o_ref):
        o_ref[...] = x_ref[:, :dim]

    return pl.pallas_call(
        body,
        grid=(batch // _BLK,),
        in_specs=[pl.BlockSpec((_BLK, _PAD_W), lambda i: (i, 0))],
        out_specs=pl.BlockSpec((_BLK, dim), lambda i: (i, 0)),
        out_shape=jax.ShapeDtypeStruct((batch, dim), jnp.float32),
    )(gathered)


def kernel(writer_ids, table, W, b):
    dim = table.shape[1]
    idx = writer_ids.astype(jnp.int32)
    table_t = jnp.swapaxes(table, 0, 1)
    w_pad = jnp.zeros((_PAD_W, dim), W.dtype).at[:dim, :].set(W)
    b_pad = jnp.zeros((1, _PAD_W), b.dtype).at[0, :dim].set(b)
    projected = _tc_project_table(table_t, w_pad, b_pad)
    gathered = _sc_gather(projected, idx)
    return _tc_slice(gathered, dim)


# speedup vs baseline: 1.3506x; 1.3506x over previous
import functools

import jax
import jax.numpy as jnp
from jax import lax
from jax.experimental import pallas as pl
from jax.experimental.pallas import tpu as pltpu
from jax.experimental.pallas import tpu_sc as plsc

_NUM_CORES = 2
_NUM_SUBCORES = 16
_NUM_WORKERS = _NUM_CORES * _NUM_SUBCORES

_BLK = 16384
_PAD_W = 128


def _tc_project_table(table_t, w_pad, b_pad):
    dim, n = table_t.shape
    grid = (n + _BLK - 1) // _BLK

    def body(x_ref, w_ref, b_ref, o_ref):
        o_ref[...] = lax.dot_general(
            x_ref[...],
            w_ref[...],
            dimension_numbers=(((0,), (1,)), ((), ())),
            preferred_element_type=jnp.float32,
            precision=lax.Precision.DEFAULT,
        ) + b_ref[...]

    return pl.pallas_call(
        body,
        grid=(grid,),
        in_specs=[
            pl.BlockSpec((dim, _BLK), lambda i: (0, i)),
            pl.BlockSpec((_PAD_W, dim), lambda i: (0, 0)),
            pl.BlockSpec((1, _PAD_W), lambda i: (0, 0)),
        ],
        out_specs=pl.BlockSpec((_BLK, _PAD_W), lambda i: (i, 0)),
        out_shape=jax.ShapeDtypeStruct((n, _PAD_W), jnp.float32),
        compiler_params=pltpu.CompilerParams(
            dimension_semantics=("parallel",),
        ),
    )(table_t, w_pad, b_pad)


def _sc_gather(projected, idx):
    batch = idx.shape[0]
    width = projected.shape[1]
    b_per_w = batch // _NUM_WORKERS
    mesh = plsc.VectorSubcoreMesh(core_axis_name="c", subcore_axis_name="s")

    @functools.partial(
        pl.kernel,
        mesh=mesh,
        out_type=jax.ShapeDtypeStruct((batch, width), projected.dtype),
        scratch_types=[
            pltpu.VMEM((b_per_w,), jnp.int32),
            pltpu.VMEM((b_per_w, width), projected.dtype),
            pltpu.SemaphoreType.DMA,
        ],
    )
    def gather_kernel(tbl_hbm, idx_hbm, out_hbm, idx_v, rows_v, sem):
        wid = lax.axis_index("s") * _NUM_CORES + lax.axis_index("c")
        base = wid * b_per_w
        pltpu.sync_copy(idx_hbm.at[pl.ds(base, b_per_w)], idx_v)
        pltpu.async_copy(tbl_hbm.at[idx_v], rows_v, sem).wait()
        pltpu.sync_copy(rows_v, out_hbm.at[pl.ds(base, b_per_w)])

    return gather_kernel(projected, idx)


def _tc_slice(gathered, dim):
    batch = gathered.shape[0]

    def body(x_ref, o_ref):
        o_ref[...] = x_ref[:, :dim]

    return pl.pallas_call(
        body,
        grid=(batch // _BLK,),
        in_specs=[pl.BlockSpec((_BLK, _PAD_W), lambda i: (i, 0))],
        out_specs=pl.BlockSpec((_BLK, dim), lambda i: (i, 0)),
        out_shape=jax.ShapeDtypeStruct((batch, dim), jnp.float32),
    )(gathered)


def kernel(writer_ids, table, W, b):
    dim = table.shape[1]
    idx = writer_ids.astype(jnp.int32)
    table_t = jnp.swapaxes(table, 0, 1)
    w_pad = jnp.zeros((_PAD_W, dim), W.dtype).at[:dim, :].set(W)
    b_pad = jnp.zeros((1, _PAD_W), b.dtype).at[0, :dim].set(b)
    projected = _tc_project_table(table_t, w_pad, b_pad)
    gathered = _sc_gather(projected, idx)
    return _tc_slice(gathered, dim)


# speedup vs baseline: 1.3860x; 1.0262x over previous
import functools

import jax
import jax.numpy as jnp
from jax import lax
from jax.experimental import pallas as pl
from jax.experimental.pallas import tpu as pltpu
from jax.experimental.pallas import tpu_sc as plsc

_NUM_CORES = 2
_NUM_SUBCORES = 16
_NUM_WORKERS = _NUM_CORES * _NUM_SUBCORES

_BLK = 32768
_PAD_W = 128


def _tc_project_table(table_t, w_pad, b_pad):
    dim, n = table_t.shape
    grid = (n + _BLK - 1) // _BLK

    def body(x_ref, w_ref, b_ref, o_ref):
        o_ref[...] = lax.dot_general(
            x_ref[...],
            w_ref[...],
            dimension_numbers=(((0,), (1,)), ((), ())),
            preferred_element_type=jnp.float32,
            precision=lax.Precision.DEFAULT,
        ) + b_ref[...]

    return pl.pallas_call(
        body,
        grid=(grid,),
        in_specs=[
            pl.BlockSpec((dim, _BLK), lambda i: (0, i)),
            pl.BlockSpec((_PAD_W, dim), lambda i: (0, 0)),
            pl.BlockSpec((1, _PAD_W), lambda i: (0, 0)),
        ],
        out_specs=pl.BlockSpec((_BLK, _PAD_W), lambda i: (i, 0)),
        out_shape=jax.ShapeDtypeStruct((n, _PAD_W), jnp.float32),
        compiler_params=pltpu.CompilerParams(
            dimension_semantics=("parallel",),
            vmem_limit_bytes=100 * 1024 * 1024,
        ),
    )(table_t, w_pad, b_pad)


def _sc_gather(projected, idx):
    batch = idx.shape[0]
    width = projected.shape[1]
    b_per_w = batch // _NUM_WORKERS
    mesh = plsc.VectorSubcoreMesh(core_axis_name="c", subcore_axis_name="s")

    @functools.partial(
        pl.kernel,
        mesh=mesh,
        out_type=jax.ShapeDtypeStruct((batch, width), projected.dtype),
        scratch_types=[
            pltpu.VMEM((b_per_w,), jnp.int32),
            pltpu.VMEM((b_per_w, width), projected.dtype),
            pltpu.SemaphoreType.DMA,
        ],
    )
    def gather_kernel(tbl_hbm, idx_hbm, out_hbm, idx_v, rows_v, sem):
        wid = lax.axis_index("s") * _NUM_CORES + lax.axis_index("c")
        base = wid * b_per_w
        pltpu.sync_copy(idx_hbm.at[pl.ds(base, b_per_w)], idx_v)
        pltpu.async_copy(tbl_hbm.at[idx_v], rows_v, sem).wait()
        pltpu.sync_copy(rows_v, out_hbm.at[pl.ds(base, b_per_w)])

    return gather_kernel(projected, idx)


def _tc_slice(gathered, dim):
    batch = gathered.shape[0]

    blk = 4096

    def body(x_ref, o_ref):
        o_ref[...] = x_ref[:, :dim]

    return pl.pallas_call(
        body,
        grid=(batch // blk,),
        in_specs=[pl.BlockSpec((blk, _PAD_W), lambda i: (i, 0))],
        out_specs=pl.BlockSpec((blk, dim), lambda i: (i, 0)),
        out_shape=jax.ShapeDtypeStruct((batch, dim), jnp.float32),
    )(gathered)


def kernel(writer_ids, table, W, b):
    dim = table.shape[1]
    idx = writer_ids.astype(jnp.int32)
    table_t = jnp.swapaxes(table, 0, 1)
    w_pad = jnp.zeros((_PAD_W, dim), W.dtype).at[:dim, :].set(W)
    b_pad = jnp.zeros((1, _PAD_W), b.dtype).at[0, :dim].set(b)
    projected = _tc_project_table(table_t, w_pad, b_pad)
    gathered = _sc_gather(projected, idx)
    return _tc_slice(gathered, dim)
